# Initial kernel scaffold; baseline (speedup 1.0000x reference)
import jax
import jax.numpy as jnp
from jax.experimental import pallas as pl


def kernel(q, k, v, e, edge_index):
    raise NotImplementedError("write your pallas kernel here")


# speedup vs baseline: 25.4656x; 25.4656x over previous
import functools

import jax
import jax.numpy as jnp
from jax import lax
from jax.experimental import pallas as pl
from jax.experimental.pallas import tpu as pltpu
from jax.experimental.pallas import tpu_sc as plsc

N = 10000
E = 320000
H = 8
C = 16
HC = H * C
ROW = HC + 16
NC = 2
NS = 16
NT = NC * NS
EPT = E // NT
B = 40
NBLK = EPT // B
NPAD = 10240
RPS = NPAD // NS
SCALE = 1.0 / (C ** 0.5)

_mesh = plsc.VectorSubcoreMesh(core_axis_name="c", subcore_axis_name="s")

_DNUMS = lax.GatherDimensionNumbers(
    offset_dims=(), collapsed_slice_dims=(0,), start_index_map=(0,))


def _take16(x, idx):
    return lax.gather(x, idx[:, None], _DNUMS, (1,),
                      mode=lax.GatherScatterMode.PROMISE_IN_BOUNDS)


@functools.partial(
    pl.kernel,
    out_type=jax.ShapeDtypeStruct((NC, NPAD, ROW), jnp.float32),
    mesh=_mesh,
    compiler_params=pltpu.CompilerParams(needs_layout_passes=False, use_tc_tiling_on_sc=False),
    scratch_types=[
        pltpu.VMEM((B,), jnp.int32),
        pltpu.VMEM((B,), jnp.int32),
        pltpu.VMEM((B, HC), jnp.float32),
        pltpu.VMEM((B, HC), jnp.float32),
        pltpu.VMEM((B, HC), jnp.float32),
        pltpu.VMEM((B, HC), jnp.float32),
        pltpu.VMEM((B, ROW), jnp.float32),
        pltpu.SemaphoreType.DMA,
        pltpu.VMEM_SHARED((NPAD, ROW), jnp.float32),
    ],
)
def _attn(qh, kh, vh, eh, srch, dsth, acch,
          sidx, didx, qb, kb, vb, eb, ob, sem, acc_sh):
    c = lax.axis_index("c")
    s = lax.axis_index("s")
    wid = c * NS + s

    lanes = lax.iota(jnp.int32, 16)
    tidx = [(lanes & 7) * C + 2 * j + (lanes >> 3) for j in range(H)]
    swap8 = lanes ^ 8
    bcast = [jnp.full((16,), j, jnp.int32) for j in range(H)]
    zero16 = jnp.zeros((16,), jnp.float32)

    def zrow(i, _):
        for j in range(ROW // 16):
            ob[i, pl.ds(16 * j, 16)] = zero16
        return 0
    lax.fori_loop(0, B, zrow, 0, unroll=False)

    nfull = RPS // B
    rem = RPS - nfull * B
    def zcopy(u, _):
        pltpu.sync_copy(ob, acc_sh.at[pl.ds(s * RPS + u * B, B)])
        return 0
    lax.fori_loop(0, nfull, zcopy, 0, unroll=False)
    if rem:
        pltpu.sync_copy(ob.at[pl.ds(0, rem)],
                        acc_sh.at[pl.ds(s * RPS + nfull * B, rem)])
    plsc.subcore_barrier()

    base_t = wid * EPT

    def blk(b, _):
        base = base_t + b * B
        pltpu.sync_copy(srch.at[pl.ds(base, B)], sidx)
        pltpu.sync_copy(dsth.at[pl.ds(base, B)], didx)
        cpe = pltpu.async_copy(eh.at[pl.ds(base, B)], eb, sem)
        cpk = pltpu.async_copy(kh.at[sidx], kb, sem)
        cpv = pltpu.async_copy(vh.at[sidx], vb, sem)
        cpq = pltpu.async_copy(qh.at[didx], qb, sem)
        cpe.wait()
        cpk.wait()
        cpv.wait()
        cpq.wait()

        def edge(i, _):
            rowv = jnp.full((16,), i, jnp.int32)
            acc = zero16
            for j in range(H):
                qt = plsc.load_gather(qb, [rowv, tidx[j]])
                kt = plsc.load_gather(kb, [rowv, tidx[j]])
                et = plsc.load_gather(eb, [rowv, tidx[j]])
                acc = acc + qt * (kt + et)
            acc = acc + _take16(acc, swap8)
            w = jnp.exp(acc * SCALE)
            for j in range(H):
                wj = _take16(w, bcast[j])
                vj = vb[i, pl.ds(C * j, C)]
                ej = eb[i, pl.ds(C * j, C)]
                ob[i, pl.ds(C * j, C)] = wj * (vj + ej)
            ob[i, pl.ds(HC, 16)] = w
            return 0
        lax.fori_loop(0, B, edge, 0, unroll=False)
        pltpu.sync_copy(ob, acc_sh.at[didx], add=True)
        return 0
    lax.fori_loop(0, NBLK, blk, 0, unroll=False)

    plsc.subcore_barrier()
    pltpu.sync_copy(acc_sh.at[pl.ds(s * RPS, RPS)],
                    acch.at[c, pl.ds(s * RPS, RPS)])


U = 40
NU = N // U


@functools.partial(
    pl.kernel,
    out_type=jax.ShapeDtypeStruct((N, HC), jnp.float32),
    mesh=_mesh,
    scratch_types=[
        pltpu.VMEM((U, ROW), jnp.float32),
        pltpu.VMEM((U, ROW), jnp.float32),
        pltpu.VMEM((U, HC), jnp.float32),
    ],
)
def _finish(acch, outh, a0, a1, ob):
    c = lax.axis_index("c")
    s = lax.axis_index("s")
    wid = c * NS + s
    nu = jnp.where(wid < 26, 8, 7)
    ubase = jnp.where(wid < 26, wid * 8, 208 + (wid - 26) * 7)
    bcast = [jnp.full((16,), j, jnp.int32) for j in range(H)]

    def unit(u, _):
        r0 = (ubase + u) * U
        pltpu.sync_copy(acch.at[0, pl.ds(r0, U)], a0)
        pltpu.sync_copy(acch.at[1, pl.ds(r0, U)], a1)

        def row(i, _):
            d = a0[i, pl.ds(HC, 16)] + a1[i, pl.ds(HC, 16)]
            for j in range(H):
                nj = a0[i, pl.ds(C * j, C)] + a1[i, pl.ds(C * j, C)]
                dj = _take16(d, bcast[j])
                ob[i, pl.ds(C * j, C)] = nj / jnp.maximum(dj, 1e-30)
            return 0
        lax.fori_loop(0, U, row, 0, unroll=False)
        pltpu.sync_copy(ob, outh.at[pl.ds(r0, U)])
        return 0
    lax.fori_loop(0, nu, unit, 0, unroll=False)


def kernel(q, k, v, e, edge_index):
    q2 = q.reshape(N, HC)
    k2 = k.reshape(N, HC)
    v2 = v.reshape(N, HC)
    e2 = e.reshape(E, HC)
    src = edge_index[0]
    dst = edge_index[1]
    acc = _attn(q2, k2, v2, e2, src, dst)
    out = _finish(acc)
    return out.reshape(N, H, C)


# speedup vs baseline: 30.2077x; 1.1862x over previous
import functools

import jax
import jax.numpy as jnp
from jax import lax
from jax.experimental import pallas as pl
from jax.experimental.pallas import tpu as pltpu
from jax.experimental.pallas import tpu_sc as plsc

N = 10000
E = 320000
H = 8
C = 16
HC = H * C
ROW = HC + 16
NC = 2
NS = 16
NT = NC * NS
EPT = E // NT
B = 16
NBLK = EPT // B
NPAD = 10240
RPS = NPAD // NS
SCALE = 1.0 / (C ** 0.5)

_mesh = plsc.VectorSubcoreMesh(core_axis_name="c", subcore_axis_name="s")

_DNUMS = lax.GatherDimensionNumbers(
    offset_dims=(), collapsed_slice_dims=(0,), start_index_map=(0,))


def _take16(x, idx):
    return lax.gather(x, idx[:, None], _DNUMS, (1,),
                      mode=lax.GatherScatterMode.PROMISE_IN_BOUNDS)


@functools.partial(
    pl.kernel,
    out_type=jax.ShapeDtypeStruct((NC, NPAD, ROW), jnp.float32),
    mesh=_mesh,
    compiler_params=pltpu.CompilerParams(needs_layout_passes=False,
                                         use_tc_tiling_on_sc=False),
    scratch_types=[
        pltpu.VMEM((2, B), jnp.int32),
        pltpu.VMEM((2, B), jnp.int32),
        pltpu.VMEM((2, B, HC), jnp.float32),
        pltpu.VMEM((2, B, HC), jnp.float32),
        pltpu.VMEM((2, B, HC), jnp.float32),
        pltpu.VMEM((2, B, HC), jnp.float32),
        pltpu.VMEM((2, B, ROW), jnp.float32),
        pltpu.SemaphoreType.DMA,
        pltpu.SemaphoreType.DMA,
        pltpu.SemaphoreType.DMA,
        pltpu.SemaphoreType.DMA,
        pltpu.VMEM_SHARED((NPAD, ROW), jnp.float32),
    ],
)
def _attn(qh, kh, vh, eh, srch, dsth, acch,
          sidx, didx, qb, kb, vb, eb, ob,
          semi0, semi1, semd0, semd1, acc_sh):
    c = lax.axis_index("c")
    s = lax.axis_index("s")
    wid = c * NS + s
    semi = (semi0, semi1)
    semd = (semd0, semd1)

    lanes = lax.iota(jnp.int32, 16)
    tidx = [(lanes & 7) * C + 2 * j + (lanes >> 3) for j in range(H)]
    swap8 = lanes ^ 8
    bcast = [jnp.full((16,), j, jnp.int32) for j in range(H)]
    zero16 = jnp.zeros((16,), jnp.float32)

    def zrow(i, _):
        for j in range(ROW // 16):
            ob[0, i, pl.ds(16 * j, 16)] = zero16
        return 0
    lax.fori_loop(0, B, zrow, 0, unroll=False)

    def zcopy(u, _):
        pltpu.sync_copy(ob.at[0], acc_sh.at[pl.ds(s * RPS + u * B, B)])
        return 0
    lax.fori_loop(0, RPS // B, zcopy, 0, unroll=False)
    plsc.subcore_barrier()

    row_t = wid * NBLK

    def fire_idx(b, par):
        pltpu.async_copy(srch.at[row_t + b], sidx.at[par], semi[par])
        pltpu.async_copy(dsth.at[row_t + b], didx.at[par], semi[par])

    def drain_idx(b, par):
        pltpu.make_async_copy(srch.at[row_t + b], sidx.at[par], semi[par]).wait()
        pltpu.make_async_copy(dsth.at[row_t + b], didx.at[par], semi[par]).wait()

    def fire_gath(b, par):
        base = wid * EPT + b * B
        sem = semd[par]
        pltpu.async_copy(eh.at[pl.ds(base, B)], eb.at[par], sem)
        pltpu.async_copy(kh.at[sidx.at[par]], kb.at[par], sem)
        pltpu.async_copy(vh.at[sidx.at[par]], vb.at[par], sem)
        pltpu.async_copy(qh.at[didx.at[par]], qb.at[par], sem)

    def drain_gath(b, par):
        base = wid * EPT + b * B
        sem = semd[par]
        pltpu.make_async_copy(eh.at[pl.ds(base, B)], eb.at[par], sem).wait()
        pltpu.make_async_copy(kh.at[sidx.at[par]], kb.at[par], sem).wait()
        pltpu.make_async_copy(vh.at[sidx.at[par]], vb.at[par], sem).wait()
        pltpu.make_async_copy(qh.at[didx.at[par]], qb.at[par], sem).wait()

    def compute(b, par):
        def edge(i, _):
            rowv = jnp.full((16,), i, jnp.int32)
            acc = zero16
            for j in range(H):
                qt = plsc.load_gather(qb.at[par], [rowv, tidx[j]])
                kt = plsc.load_gather(kb.at[par], [rowv, tidx[j]])
                et = plsc.load_gather(eb.at[par], [rowv, tidx[j]])
                acc = acc + qt * (kt + et)
            acc = acc + _take16(acc, swap8)
            w = jnp.exp(acc * SCALE)
            for j in range(H):
                wj = _take16(w, bcast[j])
                vj = vb[par, i, pl.ds(C * j, C)]
                ej = eb[par, i, pl.ds(C * j, C)]
                ob[par, i, pl.ds(C * j, C)] = wj * (vj + ej)
            ob[par, i, pl.ds(HC, 16)] = w
            return 0
        lax.fori_loop(0, B, edge, 0, unroll=False)
        pltpu.sync_copy(ob.at[par], acc_sh.at[didx.at[par]], add=True)

    fire_idx(0, 0)
    fire_idx(1, 1)
    drain_idx(0, 0)
    fire_gath(0, 0)

    @pl.loop(0, NBLK - 1, step=2)
    def _blk(b0):
        for off in range(2):
            b = b0 + off
            par = off

            @pl.when(b < NBLK - 1)
            def _():
                drain_idx(b + 1, 1 - par)
                fire_gath(b + 1, 1 - par)

            drain_gath(b, par)
            compute(b, par)

            @pl.when(b < NBLK - 2)
            def _():
                fire_idx(b + 2, par)

    drain_gath(NBLK - 1, 0)
    compute(NBLK - 1, 0)

    plsc.subcore_barrier()
    pltpu.sync_copy(acc_sh.at[pl.ds(s * RPS, RPS)],
                    acch.at[c, pl.ds(s * RPS, RPS)])


U = 40
NU = N // U


@functools.partial(
    pl.kernel,
    out_type=jax.ShapeDtypeStruct((N, HC), jnp.float32),
    mesh=_mesh,
    scratch_types=[
        pltpu.VMEM((U, ROW), jnp.float32),
        pltpu.VMEM((U, ROW), jnp.float32),
        pltpu.VMEM((U, HC), jnp.float32),
    ],
)
def _finish(acch, outh, a0, a1, ob):
    c = lax.axis_index("c")
    s = lax.axis_index("s")
    wid = c * NS + s
    nu = jnp.where(wid < 26, 8, 7)
    ubase = jnp.where(wid < 26, wid * 8, 208 + (wid - 26) * 7)
    bcast = [jnp.full((16,), j, jnp.int32) for j in range(H)]

    def unit(u, _):
        r0 = (ubase + u) * U
        pltpu.sync_copy(acch.at[0, pl.ds(r0, U)], a0)
        pltpu.sync_copy(acch.at[1, pl.ds(r0, U)], a1)

        def row(i, _):
            d = a0[i, pl.ds(HC, 16)] + a1[i, pl.ds(HC, 16)]
            for j in range(H):
                nj = a0[i, pl.ds(C * j, C)] + a1[i, pl.ds(C * j, C)]
                dj = _take16(d, bcast[j])
                ob[i, pl.ds(C * j, C)] = nj / jnp.maximum(dj, 1e-30)
            return 0
        lax.fori_loop(0, U, row, 0, unroll=False)
        pltpu.sync_copy(ob, outh.at[pl.ds(r0, U)])
        return 0
    lax.fori_loop(0, nu, unit, 0, unroll=False)


def kernel(q, k, v, e, edge_index):
    q2 = q.reshape(N, HC)
    k2 = k.reshape(N, HC)
    v2 = v.reshape(N, HC)
    e2 = e.reshape(E, HC)
    src = edge_index[0].reshape(E // B, B)
    dst = edge_index[1].reshape(E // B, B)
    acc = _attn(q2, k2, v2, e2, src, dst)
    out = _finish(acc)
    return out.reshape(N, H, C)
